# baseline (speedup 1.0000x reference)
import functools

import jax
import jax.numpy as jnp
from jax import lax
from jax.experimental import pallas as pl
from jax.experimental.pallas import tpu as pltpu
from jax.experimental.pallas import tpu_sc as plsc

WD, D32 = 128, 32
TPSUB = 3
TPD = TPSUB * D32
OUT_D = WD + TPD
NC, NS = 2, 16
NW = NC * NS
NBUF = 2
LANES = 16


@functools.partial(jax.jit, static_argnames=("T", "C", "nchunk"))
def _emb_call(widx, tpaddr, word_table, tp_flat, T, C, nchunk):
    tpw = T // NW
    ntp = tp_flat.shape[0]
    mesh = plsc.VectorSubcoreMesh(core_axis_name="c", subcore_axis_name="s")

    buf_types = []
    for _ in range(NBUF):
        buf_types += [
            pltpu.VMEM((C, WD), jnp.float32),
            pltpu.VMEM((C, TPD), jnp.float32),
            pltpu.SemaphoreType.DMA,
            pltpu.SemaphoreType.DMA,
        ]

    @functools.partial(
        pl.kernel,
        out_type=jax.ShapeDtypeStruct((T, OUT_D), jnp.float32),
        mesh=mesh,
        scratch_types=[
            pltpu.VMEM((nchunk, C), jnp.int32),
            pltpu.VMEM((nchunk, TPSUB, C), jnp.int32),
            pltpu.VMEM((ntp,), jnp.float32),
            pltpu.SemaphoreType.DMA,
        ] + buf_types,
        compiler_params=pltpu.CompilerParams(use_tc_tiling_on_sc=False, needs_layout_passes=False),
    )
    def emb(wi_hbm, tpa_hbm, wt_hbm, tpf_hbm, out_hbm,
            widx_v, tpaddr_v, tp_local, ssem, *bufs):
        slots = [bufs[4 * b:4 * b + 4] for b in range(NBUF)]
        wid = lax.axis_index("s") * NC + lax.axis_index("c")
        pltpu.async_copy(wi_hbm.at[wid], widx_v, ssem)
        pltpu.async_copy(tpa_hbm.at[wid], tpaddr_v, ssem)
        pltpu.async_copy(tpf_hbm, tp_local, ssem)
        pltpu.make_async_copy(wi_hbm.at[wid], widx_v, ssem).wait()
        pltpu.make_async_copy(tpa_hbm.at[wid], tpaddr_v, ssem).wait()
        pltpu.make_async_copy(tpf_hbm, tp_local, ssem).wait()

        def fire_word(i, b):
            wbuf, _, gsem, _ = slots[b]
            pltpu.async_copy(wt_hbm.at[widx_v.at[i]], wbuf, gsem)

        def drain_word(i, b):
            wbuf, _, gsem, _ = slots[b]
            pltpu.make_async_copy(wt_hbm.at[widx_v.at[i]], wbuf, gsem).wait()

        def assemble_tp(i, b):
            _, tpbuf, _, _ = slots[b]
            lane = lax.iota(jnp.int32, LANES)

            @pl.loop(0, C // LANES)
            def grp(g):
                t0 = g * LANES
                for k in range(TPSUB):
                    src = plsc.load_gather(
                        tpaddr_v,
                        [jnp.full((LANES,), i, jnp.int32),
                         jnp.full((LANES,), k, jnp.int32),
                         t0 + lane])
                    for j in range(D32):
                        v = plsc.load_gather(tp_local, [src + j])
                        plsc.store_scatter(
                            tpbuf,
                            [t0 + lane,
                             jnp.full((LANES,), k * D32 + j, jnp.int32)], v)

        def fire_writes(i, b):
            wbuf, tpbuf, _, wsem = slots[b]
            base = wid * tpw + i * C
            pltpu.async_copy(wbuf, out_hbm.at[pl.ds(base, C), pl.ds(0, WD)], wsem)
            pltpu.async_copy(tpbuf,
                             out_hbm.at[pl.ds(base, C), pl.ds(WD, TPD)], wsem)

        def drain_writes(i, b):
            wbuf, tpbuf, _, wsem = slots[b]
            base = wid * tpw + i * C
            pltpu.make_async_copy(
                wbuf, out_hbm.at[pl.ds(base, C), pl.ds(0, WD)], wsem).wait()
            pltpu.make_async_copy(
                tpbuf, out_hbm.at[pl.ds(base, C), pl.ds(WD, TPD)], wsem).wait()

        fire_word(0, 0)
        fire_word(1, 1)
        assemble_tp(0, 0)

        @pl.loop(0, nchunk // NBUF)
        def body(j):
            for b in range(NBUF):
                i = j * NBUF + b
                drain_word(i, b)
                fire_writes(i, b)

                @pl.when(i + 1 < nchunk)
                def _():
                    sb = (b + 1) % NBUF

                    @pl.when(i >= 1)
                    def _():
                        drain_writes(i - 1, sb)

                    assemble_tp(i + 1, sb)

                @pl.when(i + NBUF < nchunk)
                def _():
                    fire_word(i + NBUF, b)

        drain_writes(nchunk - 1, (nchunk - 1) % NBUF)

    return emb(widx, tpaddr, word_table, tp_flat)


def kernel(word_id, tag_id, pos_1, pos_2, word_table, tag_table, pos_table):
    B, L = word_id.shape
    T = B * L
    C = 128
    nchunk = T // (NW * C)
    ntag = tag_table.shape[0]
    tp_flat = jnp.concatenate([tag_table, pos_table], axis=0).reshape(-1)
    shape = (NW, nchunk, C)
    tpaddr = jnp.stack([
        tag_id.reshape(shape).astype(jnp.int32) * D32,
        (pos_1.reshape(shape).astype(jnp.int32) + ntag) * D32,
        (pos_2.reshape(shape).astype(jnp.int32) + ntag) * D32,
    ], axis=2)
    out = _emb_call(
        word_id.reshape(shape).astype(jnp.int32),
        tpaddr, word_table, tp_flat,
        T=T, C=C, nchunk=nchunk,
    )
    return out.reshape(B, L, OUT_D)
